# Initial kernel scaffold; baseline (speedup 1.0000x reference)
import jax
import jax.numpy as jnp
from jax.experimental import pallas as pl


def kernel(x, edge_index, edge_attr, W_in, b_in, W_e, b_e, root, conv_b, W_out, b_out):
    raise NotImplementedError("write your pallas kernel here")


# speedup vs baseline: 1.3253x; 1.3253x over previous
import functools

import jax
import jax.numpy as jnp
from jax import lax
from jax.experimental import pallas as pl
from jax.experimental.pallas import tpu as pltpu
from jax.experimental.pallas import tpu_sc as plsc

N = 10000
E = 160000
D_IN = 128
D_EDGE = 16
C = 16

NC = 2
NS = 16
NW = NC * NS

CH = 128
BPW = 5120
EP = NW * BPW
NCHUNK = BPW // CH
NA = 10240
RPT = NA // NS

BN = 2000
BE = 2048


def _lrelu(v):
    return jnp.where(v > 0, v, 0.01 * v)


def _node_body(x_ref, w_in_ref, b_in_ref, root_ref, conv_b_ref, h_ref, hr_ref):
    h = _lrelu(jnp.dot(x_ref[...], w_in_ref[...],
                       preferred_element_type=jnp.float32) + b_in_ref[...])
    h_ref[...] = h
    hr_ref[...] = jnp.dot(h, root_ref[...],
                          preferred_element_type=jnp.float32) + conv_b_ref[...]


def _node_stage(x, W_in, b_in, root, conv_b):
    grid = (N // BN,)
    return pl.pallas_call(
        _node_body,
        grid=grid,
        in_specs=[
            pl.BlockSpec((BN, D_IN), lambda i: (i, 0)),
            pl.BlockSpec((D_IN, C), lambda i: (0, 0)),
            pl.BlockSpec((1, C), lambda i: (0, 0)),
            pl.BlockSpec((C, C), lambda i: (0, 0)),
            pl.BlockSpec((1, C), lambda i: (0, 0)),
        ],
        out_specs=[
            pl.BlockSpec((BN, C), lambda i: (i, 0)),
            pl.BlockSpec((BN, C), lambda i: (i, 0)),
        ],
        out_shape=[
            jax.ShapeDtypeStruct((N, C), jnp.float32),
            jax.ShapeDtypeStruct((N, C), jnp.float32),
        ],
    )(x, W_in, b_in.reshape(1, C), root, conv_b.reshape(1, C))


def _gather_body(h_hbm, src_hbm, out_hbm, idx_v, rows_v, sem):
    cid = lax.axis_index("c")
    sid = lax.axis_index("s")
    wid = sid * NC + cid
    pltpu.sync_copy(src_hbm.at[pl.ds(wid * NCHUNK, NCHUNK)], idx_v)

    def body(it, carry):
        cps = []
        for i in range(8):
            j = it * 8 + i
            cps.append(pltpu.async_copy(
                h_hbm.at[idx_v.at[j]], rows_v.at[pl.ds(j * CH, CH)], sem))
        for cp in cps:
            cp.wait()
        return carry

    lax.fori_loop(0, NCHUNK // 8, body, 0)
    pltpu.sync_copy(rows_v, out_hbm.at[pl.ds(wid * BPW, BPW)])


def _edge_body(ea_ref, hs_ref, w_e_ref, b_e_ref, msg_ref):
    z = jnp.dot(ea_ref[...], w_e_ref[...],
                preferred_element_type=jnp.float32) + b_e_ref[...]
    w = _lrelu(z)
    hs = hs_ref[...]
    acc = hs[:, 0:1] * w[:, 0:C]
    for c in range(1, C):
        acc = acc + hs[:, c:c + 1] * w[:, c * C:(c + 1) * C]
    msg_ref[...] = acc


def _edge_stage(ea_p, h_src, W_e, b_e2):
    grid = (EP // BE,)
    return pl.pallas_call(
        _edge_body,
        grid=grid,
        in_specs=[
            pl.BlockSpec((BE, D_EDGE), lambda i: (i, 0)),
            pl.BlockSpec((BE, C), lambda i: (i, 0)),
            pl.BlockSpec((D_EDGE, C * C), lambda i: (0, 0)),
            pl.BlockSpec((1, C * C), lambda i: (0, 0)),
        ],
        out_specs=pl.BlockSpec((BE, C), lambda i: (i, 0)),
        out_shape=jax.ShapeDtypeStruct((EP, C), jnp.float32),
    )(ea_p, h_src, W_e, b_e2)


def _scatter_body(msg_hbm, dst_hbm, out_hbm, idx_v, msg_v, zero_v, aggr_sh, sem):
    cid = lax.axis_index("c")
    sid = lax.axis_index("s")
    wid = sid * NC + cid
    pltpu.sync_copy(dst_hbm.at[pl.ds(wid * NCHUNK, NCHUNK)], idx_v)
    pltpu.sync_copy(msg_hbm.at[pl.ds(wid * BPW, BPW)], msg_v)

    zv = jnp.zeros((C,), jnp.float32)

    def zbody(i, carry):
        for r in range(8):
            zero_v[i * 8 + r, :] = zv
        return carry

    lax.fori_loop(0, RPT // 8, zbody, 0)
    pltpu.sync_copy(zero_v, aggr_sh.at[pl.ds(sid * RPT, RPT)])
    plsc.subcore_barrier()

    def body(j, carry):
        pltpu.sync_copy(msg_v.at[pl.ds(j * CH, CH)], aggr_sh.at[idx_v.at[j]],
                        add=True)
        return carry

    lax.fori_loop(0, NCHUNK, body, 0)
    plsc.subcore_barrier()
    pltpu.sync_copy(aggr_sh.at[pl.ds(sid * RPT, RPT)],
                    out_hbm.at[pl.ds(cid * NA + sid * RPT, RPT)])


def _final_body(p0_ref, p1_ref, hr_ref, w_out_ref, b_out_ref, out_ref):
    hfin = p0_ref[...] + p1_ref[...] + hr_ref[...]
    z = jnp.sum(hfin * w_out_ref[...], axis=1, keepdims=True) + b_out_ref[...]
    lse = jnp.maximum(z, 0.0) + jnp.log(1.0 + jnp.exp(-jnp.abs(z)))
    out_ref[...] = jnp.concatenate([-lse, z - lse], axis=1)


def _final_stage(p0, p1, hr, W_out, b_out):
    grid = (N // BN,)
    return pl.pallas_call(
        _final_body,
        grid=grid,
        in_specs=[
            pl.BlockSpec((BN, C), lambda i: (i, 0)),
            pl.BlockSpec((BN, C), lambda i: (i, 0)),
            pl.BlockSpec((BN, C), lambda i: (i, 0)),
            pl.BlockSpec((1, C), lambda i: (0, 0)),
            pl.BlockSpec((1, 1), lambda i: (0, 0)),
        ],
        out_specs=pl.BlockSpec((BN, 2), lambda i: (i, 0)),
        out_shape=jax.ShapeDtypeStruct((N, 2), jnp.float32),
    )(p0, p1, hr, W_out.reshape(1, C), b_out.reshape(1, 1))


@functools.lru_cache(maxsize=1)
def _sc_kernels():
    mesh = plsc.VectorSubcoreMesh(core_axis_name="c", subcore_axis_name="s",
                                  num_cores=NC, num_subcores=NS)
    params = pltpu.CompilerParams(use_tc_tiling_on_sc=False)
    gather = pl.kernel(
        _gather_body,
        out_type=jax.ShapeDtypeStruct((EP, C), jnp.float32),
        mesh=mesh,
        compiler_params=params,
        scratch_types=[
            pltpu.VMEM((NCHUNK, CH), jnp.int32),
            pltpu.VMEM((BPW, C), jnp.float32),
            pltpu.SemaphoreType.DMA,
        ],
    )
    scatter = pl.kernel(
        _scatter_body,
        out_type=jax.ShapeDtypeStruct((NC * NA, C), jnp.float32),
        mesh=mesh,
        compiler_params=params,
        scratch_types=[
            pltpu.VMEM((NCHUNK, CH), jnp.int32),
            pltpu.VMEM((BPW, C), jnp.float32),
            pltpu.VMEM((RPT, C), jnp.float32),
            pltpu.VMEM_SHARED((NA, C), jnp.float32),
            pltpu.SemaphoreType.DMA,
        ],
    )
    return gather, scatter


def kernel(x, edge_index, edge_attr, W_in, b_in, W_e, b_e, root, conv_b, W_out, b_out):
    src = edge_index[0]
    dst = edge_index[1]
    pad = EP - E
    src_p = jnp.concatenate([src, jnp.zeros((pad,), jnp.int32)]).reshape(EP // CH, CH)
    dst_p = jnp.concatenate([dst, jnp.full((pad,), N, jnp.int32)]).reshape(EP // CH, CH)
    ea_p = jnp.concatenate([edge_attr, jnp.zeros((pad, D_EDGE), jnp.float32)])

    gather_k, scatter_k = _sc_kernels()
    h, hr = _node_stage(x, W_in, b_in, root, conv_b)
    h_src = gather_k(h, src_p)
    msg = _edge_stage(ea_p, h_src, W_e, b_e.reshape(1, C * C))
    partials = scatter_k(msg, dst_p)
    p0 = lax.slice(partials, (0, 0), (N, C))
    p1 = lax.slice(partials, (NA, 0), (NA + N, C))
    return _final_stage(p0, p1, hr, W_out, b_out)


# speedup vs baseline: 3.0938x; 2.3344x over previous
import functools

import jax
import jax.numpy as jnp
from jax import lax
from jax.experimental import pallas as pl
from jax.experimental.pallas import tpu as pltpu
from jax.experimental.pallas import tpu_sc as plsc

N = 10000
E = 160000
D_IN = 128
D_EDGE = 16
C = 16

NC = 2
NS = 16
NW = NC * NS

CH = 128
BPW = 5120
EP = NW * BPW
NCHUNK = BPW // CH
NA = 10240
RPT = NA // NS

BN = 2000
BE = 2048


def _lrelu(v):
    return jnp.where(v > 0, v, 0.01 * v)


def _node_body(x_ref, w_in_ref, b_in_ref, root_ref, conv_b_ref, h_ref, hr_ref):
    h = _lrelu(jnp.dot(x_ref[...], w_in_ref[...],
                       preferred_element_type=jnp.float32) + b_in_ref[...])
    h_ref[...] = h
    hr_ref[...] = jnp.dot(h, root_ref[...],
                          preferred_element_type=jnp.float32) + conv_b_ref[...]


def _node_stage(x, W_in, b_in, root, conv_b):
    grid = (N // BN,)
    return pl.pallas_call(
        _node_body,
        grid=grid,
        in_specs=[
            pl.BlockSpec((BN, D_IN), lambda i: (i, 0)),
            pl.BlockSpec((D_IN, C), lambda i: (0, 0)),
            pl.BlockSpec((1, C), lambda i: (0, 0)),
            pl.BlockSpec((C, C), lambda i: (0, 0)),
            pl.BlockSpec((1, C), lambda i: (0, 0)),
        ],
        out_specs=[
            pl.BlockSpec((BN, C), lambda i: (i, 0)),
            pl.BlockSpec((BN, C), lambda i: (i, 0)),
        ],
        out_shape=[
            jax.ShapeDtypeStruct((N, C), jnp.float32),
            jax.ShapeDtypeStruct((N, C), jnp.float32),
        ],
    )(x, W_in, b_in.reshape(1, C), root, conv_b.reshape(1, C))


def _gather_body(h_hbm, src_hbm, out_hbm, idx_v, rows_v, sem):
    cid = lax.axis_index("c")
    sid = lax.axis_index("s")
    wid = sid * NC + cid
    pltpu.sync_copy(src_hbm.at[pl.ds(wid * NCHUNK, NCHUNK)], idx_v)

    def body(it, carry):
        cps = []
        for i in range(8):
            j = it * 8 + i
            cps.append(pltpu.async_copy(
                h_hbm.at[idx_v.at[j]], rows_v.at[pl.ds(j * CH, CH)], sem))
        for cp in cps:
            cp.wait()
        return carry

    lax.fori_loop(0, NCHUNK // 8, body, 0)
    pltpu.sync_copy(rows_v, out_hbm.at[pl.ds(wid * BPW, BPW)])


def _edge_body(ea_ref, hs_ref, w_e_ref, b_e_ref, msg_ref):
    z = jnp.dot(ea_ref[...], w_e_ref[...],
                preferred_element_type=jnp.float32) + b_e_ref[...]
    w = _lrelu(z)
    hs = hs_ref[...]
    cc = lax.broadcasted_iota(jnp.int32, (C, C * C), 0)
    jj = lax.broadcasted_iota(jnp.int32, (C, C * C), 1)
    expand = (jj // C == cc).astype(jnp.float32)
    jj2 = lax.broadcasted_iota(jnp.int32, (C * C, C), 0)
    dd = lax.broadcasted_iota(jnp.int32, (C * C, C), 1)
    reduce = (jj2 % C == dd).astype(jnp.float32)
    hrep = jnp.dot(hs, expand, preferred_element_type=jnp.float32)
    msg_ref[...] = jnp.dot(w * hrep, reduce,
                           preferred_element_type=jnp.float32)


def _edge_stage(ea_p, h_src, W_e, b_e2):
    grid = (EP // BE,)
    return pl.pallas_call(
        _edge_body,
        grid=grid,
        in_specs=[
            pl.BlockSpec((BE, D_EDGE), lambda i: (i, 0)),
            pl.BlockSpec((BE, C), lambda i: (i, 0)),
            pl.BlockSpec((D_EDGE, C * C), lambda i: (0, 0)),
            pl.BlockSpec((1, C * C), lambda i: (0, 0)),
        ],
        out_specs=pl.BlockSpec((BE, C), lambda i: (i, 0)),
        out_shape=jax.ShapeDtypeStruct((EP, C), jnp.float32),
    )(ea_p, h_src, W_e, b_e2)


def _scatter_body(msg_hbm, dst_hbm, out_hbm, idx_v, msg_v, zero_v, aggr_sh, sem):
    cid = lax.axis_index("c")
    sid = lax.axis_index("s")
    wid = sid * NC + cid
    pltpu.sync_copy(dst_hbm.at[pl.ds(wid * NCHUNK, NCHUNK)], idx_v)
    pltpu.sync_copy(msg_hbm.at[pl.ds(wid * BPW, BPW)], msg_v)

    zv = jnp.zeros((C,), jnp.float32)

    def zbody(i, carry):
        for r in range(8):
            zero_v[i * 8 + r, :] = zv
        return carry

    lax.fori_loop(0, RPT // 8, zbody, 0)
    pltpu.sync_copy(zero_v, aggr_sh.at[pl.ds(sid * RPT, RPT)])
    plsc.subcore_barrier()

    def body(j, carry):
        pltpu.sync_copy(msg_v.at[pl.ds(j * CH, CH)], aggr_sh.at[idx_v.at[j]],
                        add=True)
        return carry

    lax.fori_loop(0, NCHUNK, body, 0)
    plsc.subcore_barrier()
    pltpu.sync_copy(aggr_sh.at[pl.ds(sid * RPT, RPT)],
                    out_hbm.at[pl.ds(cid * NA + sid * RPT, RPT)])


def _final_body(p0_ref, p1_ref, hr_ref, w_out_ref, b_out_ref, out_ref):
    hfin = p0_ref[...] + p1_ref[...] + hr_ref[...]
    z = jnp.sum(hfin * w_out_ref[...], axis=1, keepdims=True) + b_out_ref[...]
    lse = jnp.maximum(z, 0.0) + jnp.log(1.0 + jnp.exp(-jnp.abs(z)))
    out_ref[...] = jnp.concatenate([-lse, z - lse], axis=1)


def _final_stage(p0, p1, hr, W_out, b_out):
    grid = (N // BN,)
    return pl.pallas_call(
        _final_body,
        grid=grid,
        in_specs=[
            pl.BlockSpec((BN, C), lambda i: (i, 0)),
            pl.BlockSpec((BN, C), lambda i: (i, 0)),
            pl.BlockSpec((BN, C), lambda i: (i, 0)),
            pl.BlockSpec((1, C), lambda i: (0, 0)),
            pl.BlockSpec((1, 1), lambda i: (0, 0)),
        ],
        out_specs=pl.BlockSpec((BN, 2), lambda i: (i, 0)),
        out_shape=jax.ShapeDtypeStruct((N, 2), jnp.float32),
    )(p0, p1, hr, W_out.reshape(1, C), b_out.reshape(1, 1))


@functools.lru_cache(maxsize=1)
def _sc_kernels():
    mesh = plsc.VectorSubcoreMesh(core_axis_name="c", subcore_axis_name="s",
                                  num_cores=NC, num_subcores=NS)
    params = pltpu.CompilerParams(use_tc_tiling_on_sc=False)
    gather = pl.kernel(
        _gather_body,
        out_type=jax.ShapeDtypeStruct((EP, C), jnp.float32),
        mesh=mesh,
        compiler_params=params,
        scratch_types=[
            pltpu.VMEM((NCHUNK, CH), jnp.int32),
            pltpu.VMEM((BPW, C), jnp.float32),
            pltpu.SemaphoreType.DMA,
        ],
    )
    scatter = pl.kernel(
        _scatter_body,
        out_type=jax.ShapeDtypeStruct((NC * NA, C), jnp.float32),
        mesh=mesh,
        compiler_params=params,
        scratch_types=[
            pltpu.VMEM((NCHUNK, CH), jnp.int32),
            pltpu.VMEM((BPW, C), jnp.float32),
            pltpu.VMEM((RPT, C), jnp.float32),
            pltpu.VMEM_SHARED((NA, C), jnp.float32),
            pltpu.SemaphoreType.DMA,
        ],
    )
    return gather, scatter


def kernel(x, edge_index, edge_attr, W_in, b_in, W_e, b_e, root, conv_b, W_out, b_out):
    src = edge_index[0]
    dst = edge_index[1]
    pad = EP - E
    src_p = jnp.concatenate([src, jnp.zeros((pad,), jnp.int32)]).reshape(EP // CH, CH)
    dst_p = jnp.concatenate([dst, jnp.full((pad,), N, jnp.int32)]).reshape(EP // CH, CH)
    ea_p = jnp.concatenate([edge_attr, jnp.zeros((pad, D_EDGE), jnp.float32)])

    gather_k, scatter_k = _sc_kernels()
    h, hr = _node_stage(x, W_in, b_in, root, conv_b)
    h_src = gather_k(h, src_p)
    msg = _edge_stage(ea_p, h_src, W_e, b_e.reshape(1, C * C))
    partials = scatter_k(msg, dst_p)
    p0 = lax.slice(partials, (0, 0), (N, C))
    p1 = lax.slice(partials, (NA, 0), (NA + N, C))
    return _final_stage(p0, p1, hr, W_out, b_out)


# speedup vs baseline: 3.5798x; 1.1571x over previous
import functools

import jax
import jax.numpy as jnp
from jax import lax
from jax.experimental import pallas as pl
from jax.experimental.pallas import tpu as pltpu
from jax.experimental.pallas import tpu_sc as plsc

N = 10000
E = 160000
D_IN = 128
D_EDGE = 16
C = 16

NC = 2
NS = 16
NW = NC * NS

CH = 128
BPW = 5120
EP = NW * BPW
NCHUNK = BPW // CH
NA = 10240
RPT = NA // NS

BN = 2000
BE = 4096


def _lrelu(v):
    return jnp.where(v > 0, v, 0.01 * v)


def _node_body(x_ref, w_in_ref, b_in_ref, root_ref, conv_b_ref, h_ref, hr_ref):
    h = _lrelu(jnp.dot(x_ref[...], w_in_ref[...],
                       preferred_element_type=jnp.float32) + b_in_ref[...])
    h_ref[...] = h
    hr_ref[...] = jnp.dot(h, root_ref[...],
                          preferred_element_type=jnp.float32) + conv_b_ref[...]


def _node_stage(x, W_in, b_in, root, conv_b):
    grid = (N // BN,)
    return pl.pallas_call(
        _node_body,
        grid=grid,
        in_specs=[
            pl.BlockSpec((BN, D_IN), lambda i: (i, 0)),
            pl.BlockSpec((D_IN, C), lambda i: (0, 0)),
            pl.BlockSpec((1, C), lambda i: (0, 0)),
            pl.BlockSpec((C, C), lambda i: (0, 0)),
            pl.BlockSpec((1, C), lambda i: (0, 0)),
        ],
        out_specs=[
            pl.BlockSpec((BN, C), lambda i: (i, 0)),
            pl.BlockSpec((BN, C), lambda i: (i, 0)),
        ],
        out_shape=[
            jax.ShapeDtypeStruct((N, C), jnp.float32),
            jax.ShapeDtypeStruct((N, C), jnp.float32),
        ],
    )(x, W_in, b_in.reshape(1, C), root, conv_b.reshape(1, C))


def _gather_body(h_hbm, src_hbm, out_hbm, idx_v, rows_v, sem):
    cid = lax.axis_index("c")
    sid = lax.axis_index("s")
    wid = sid * NC + cid
    pltpu.sync_copy(src_hbm.at[pl.ds(wid * BPW, BPW)], idx_v)

    def fire(it, carry):
        for i in range(8):
            j = it * 8 + i
            pltpu.async_copy(
                h_hbm.at[idx_v.at[pl.ds(j * CH, CH)]],
                rows_v.at[pl.ds(j * CH, CH)], sem)
        return carry

    lax.fori_loop(0, NCHUNK // 8, fire, 0)
    pltpu.make_async_copy(h_hbm.at[pl.ds(0, BPW)], rows_v, sem).wait()
    pltpu.sync_copy(rows_v, out_hbm.at[pl.ds(wid * BPW, BPW)])


def _edge_body(ea_ref, hs_ref, w_e_ref, b_e_ref, msg_ref):
    z = jnp.dot(ea_ref[...], w_e_ref[...],
                preferred_element_type=jnp.float32) + b_e_ref[...]
    w = _lrelu(z)
    hs = hs_ref[...]
    cc = lax.broadcasted_iota(jnp.int32, (C, C * C), 0)
    jj = lax.broadcasted_iota(jnp.int32, (C, C * C), 1)
    expand = (jj // C == cc).astype(jnp.float32)
    jj2 = lax.broadcasted_iota(jnp.int32, (C * C, C), 0)
    dd = lax.broadcasted_iota(jnp.int32, (C * C, C), 1)
    reduce = (jj2 % C == dd).astype(jnp.float32)
    hrep = jnp.dot(hs, expand, preferred_element_type=jnp.float32)
    msg_ref[...] = jnp.dot(w * hrep, reduce,
                           preferred_element_type=jnp.float32)


def _edge_stage(ea, h_src, W_e, b_e2):
    grid = (EP // BE,)
    return pl.pallas_call(
        _edge_body,
        grid=grid,
        in_specs=[
            pl.BlockSpec((BE, D_EDGE), lambda i: (i, 0)),
            pl.BlockSpec((BE, C), lambda i: (i, 0)),
            pl.BlockSpec((D_EDGE, C * C), lambda i: (0, 0)),
            pl.BlockSpec((1, C * C), lambda i: (0, 0)),
        ],
        out_specs=pl.BlockSpec((BE, C), lambda i: (i, 0)),
        out_shape=jax.ShapeDtypeStruct((EP, C), jnp.float32),
    )(ea, h_src, W_e, b_e2)


def _scatter_body(msg_hbm, dst_hbm, out_hbm, idx_v, msg_v, zero_v, aggr_sh, sem,
                  sem_idx):
    cid = lax.axis_index("c")
    sid = lax.axis_index("s")
    wid = sid * NC + cid

    def idx_fire(it, carry):
        for i in range(4):
            j = it * 4 + i
            pltpu.async_copy(dst_hbm.at[pl.ds(wid * BPW + j * CH, CH)],
                             idx_v.at[j], sem_idx)
        return carry

    lax.fori_loop(0, NCHUNK // 4, idx_fire, 0)
    pltpu.async_copy(msg_hbm.at[pl.ds(wid * BPW, BPW)], msg_v, sem)

    zv = jnp.zeros((C,), jnp.float32)

    def zbody(i, carry):
        for r in range(8):
            zero_v[i * 8 + r, :] = zv
        return carry

    lax.fori_loop(0, RPT // 8, zbody, 0)

    def idx_drain(it, carry):
        for i in range(4):
            j = it * 4 + i
            pltpu.make_async_copy(dst_hbm.at[pl.ds(wid * BPW + j * CH, CH)],
                                  idx_v.at[j], sem_idx).wait()
        return carry

    lax.fori_loop(0, NCHUNK // 4, idx_drain, 0)
    pltpu.make_async_copy(msg_hbm.at[pl.ds(0, BPW)], msg_v, sem).wait()
    pltpu.sync_copy(zero_v, aggr_sh.at[pl.ds(sid * RPT, RPT)])
    plsc.subcore_barrier()

    def fire(it, carry):
        for i in range(8):
            j = it * 8 + i
            pltpu.async_copy(msg_v.at[pl.ds(j * CH, CH)],
                             aggr_sh.at[idx_v.at[j]], sem, add=True)
        return carry

    lax.fori_loop(0, NCHUNK // 8, fire, 0)
    pltpu.make_async_copy(msg_v, aggr_sh.at[pl.ds(0, BPW)], sem).wait()
    plsc.subcore_barrier()
    pltpu.sync_copy(aggr_sh.at[pl.ds(sid * RPT, RPT)],
                    out_hbm.at[pl.ds(cid * NA + sid * RPT, RPT)])


def _final_body(p0_ref, p1_ref, hr_ref, w_out_ref, b_out_ref, out_ref):
    hfin = p0_ref[...] + p1_ref[...] + hr_ref[...]
    z = jnp.sum(hfin * w_out_ref[...], axis=1, keepdims=True) + b_out_ref[...]
    lse = jnp.maximum(z, 0.0) + jnp.log(1.0 + jnp.exp(-jnp.abs(z)))
    out_ref[...] = jnp.concatenate([-lse, z - lse], axis=1)


def _final_stage(p0, p1, hr, W_out, b_out):
    grid = (N // BN,)
    return pl.pallas_call(
        _final_body,
        grid=grid,
        in_specs=[
            pl.BlockSpec((BN, C), lambda i: (i, 0)),
            pl.BlockSpec((BN, C), lambda i: (i, 0)),
            pl.BlockSpec((BN, C), lambda i: (i, 0)),
            pl.BlockSpec((1, C), lambda i: (0, 0)),
            pl.BlockSpec((1, 1), lambda i: (0, 0)),
        ],
        out_specs=pl.BlockSpec((BN, 2), lambda i: (i, 0)),
        out_shape=jax.ShapeDtypeStruct((N, 2), jnp.float32),
    )(p0, p1, hr, W_out.reshape(1, C), b_out.reshape(1, 1))


@functools.lru_cache(maxsize=1)
def _sc_kernels():
    mesh = plsc.VectorSubcoreMesh(core_axis_name="c", subcore_axis_name="s",
                                  num_cores=NC, num_subcores=NS)
    params = pltpu.CompilerParams(use_tc_tiling_on_sc=False)
    gather = pl.kernel(
        _gather_body,
        out_type=jax.ShapeDtypeStruct((EP, C), jnp.float32),
        mesh=mesh,
        compiler_params=params,
        scratch_types=[
            pltpu.VMEM((BPW,), jnp.int32),
            pltpu.VMEM((BPW, C), jnp.float32),
            pltpu.SemaphoreType.DMA,
        ],
    )
    scatter = pl.kernel(
        _scatter_body,
        out_type=jax.ShapeDtypeStruct((NC * NA, C), jnp.float32),
        mesh=mesh,
        compiler_params=params,
        scratch_types=[
            pltpu.VMEM((NCHUNK, CH), jnp.int32),
            pltpu.VMEM((BPW, C), jnp.float32),
            pltpu.VMEM((RPT, C), jnp.float32),
            pltpu.VMEM_SHARED((NA, C), jnp.float32),
            pltpu.SemaphoreType.DMA,
            pltpu.SemaphoreType.DMA,
        ],
    )
    return gather, scatter


def kernel(x, edge_index, edge_attr, W_in, b_in, W_e, b_e, root, conv_b, W_out, b_out):
    src = edge_index[0]
    dst = edge_index[1]
    pad = EP - E
    src_p = jnp.concatenate([src, jnp.zeros((pad,), jnp.int32)])
    dst_p = jnp.concatenate([dst, jnp.full((pad,), N, jnp.int32)])

    gather_k, scatter_k = _sc_kernels()
    h, hr = _node_stage(x, W_in, b_in, root, conv_b)
    h_src = gather_k(h, src_p)
    msg = _edge_stage(edge_attr, h_src, W_e, b_e.reshape(1, C * C))
    partials = scatter_k(msg, dst_p)
    p0 = lax.slice(partials, (0, 0), (N, C))
    p1 = lax.slice(partials, (NA, 0), (NA + N, C))
    return _final_stage(p0, p1, hr, W_out, b_out)


# speedup vs baseline: 4.0104x; 1.1203x over previous
import functools

import jax
import jax.numpy as jnp
from jax import lax
from jax.experimental import pallas as pl
from jax.experimental.pallas import tpu as pltpu
from jax.experimental.pallas import tpu_sc as plsc

N = 10000
E = 160000
D_IN = 128
D_EDGE = 16
C = 16

NC = 2
NS = 16
NW = NC * NS

CH = 128
BPW = 5120
EP = NW * BPW
NCHUNK = BPW // CH
NA = 10240
RPT = NA // NS

BN = 2000
BE = 4096


def _lrelu(v):
    return jnp.where(v > 0, v, 0.01 * v)


def _node_body(x_ref, w_in_ref, b_in_ref, root_ref, conv_b_ref, h_ref, hr_ref):
    h = _lrelu(jnp.dot(x_ref[...], w_in_ref[...],
                       preferred_element_type=jnp.float32) + b_in_ref[...])
    h_ref[...] = h
    hr_ref[...] = jnp.dot(h, root_ref[...],
                          preferred_element_type=jnp.float32) + conv_b_ref[...]


def _node_stage(x, W_in, b_in, root, conv_b):
    grid = (N // BN,)
    return pl.pallas_call(
        _node_body,
        grid=grid,
        in_specs=[
            pl.BlockSpec((BN, D_IN), lambda i: (i, 0)),
            pl.BlockSpec((D_IN, C), lambda i: (0, 0)),
            pl.BlockSpec((1, C), lambda i: (0, 0)),
            pl.BlockSpec((C, C), lambda i: (0, 0)),
            pl.BlockSpec((1, C), lambda i: (0, 0)),
        ],
        out_specs=[
            pl.BlockSpec((BN, C), lambda i: (i, 0)),
            pl.BlockSpec((BN, C), lambda i: (i, 0)),
        ],
        out_shape=[
            jax.ShapeDtypeStruct((N, C), jnp.float32),
            jax.ShapeDtypeStruct((N, C), jnp.float32),
        ],
    )(x, W_in, b_in.reshape(1, C), root, conv_b.reshape(1, C))


def _gather_body(h_hbm, src_hbm, out_hbm, idx_v, rows_v, sem, sem_idx):
    cid = lax.axis_index("c")
    sid = lax.axis_index("s")
    wid = sid * NC + cid

    def idx_fire(it, carry):
        for i in range(4):
            j = it * 4 + i
            pltpu.async_copy(src_hbm.at[pl.ds(wid * BPW + j * CH, CH)],
                             idx_v.at[j], sem_idx)
        return carry

    lax.fori_loop(0, NCHUNK // 4, idx_fire, 0)

    def idx_drain(it, carry):
        for i in range(4):
            j = it * 4 + i
            pltpu.make_async_copy(src_hbm.at[pl.ds(wid * BPW + j * CH, CH)],
                                  idx_v.at[j], sem_idx).wait()
        return carry

    lax.fori_loop(0, NCHUNK // 4, idx_drain, 0)

    def fire(it, carry):
        for i in range(8):
            j = it * 8 + i
            pltpu.async_copy(
                h_hbm.at[idx_v.at[j]],
                rows_v.at[pl.ds(j * CH, CH)], sem)
        return carry

    lax.fori_loop(0, NCHUNK // 8, fire, 0)
    pltpu.make_async_copy(h_hbm.at[pl.ds(0, BPW)], rows_v, sem).wait()
    pltpu.sync_copy(rows_v, out_hbm.at[pl.ds(wid * BPW, BPW)])


def _edge_body(eaT_ref, hs_ref, w_e_ref, b_e_ref, msg_ref):
    z = lax.dot_general(eaT_ref[...], w_e_ref[...], (((0,), (0,)), ((), ())),
                        preferred_element_type=jnp.float32) + b_e_ref[...]
    w = _lrelu(z)
    hs = hs_ref[...]
    cc = lax.broadcasted_iota(jnp.int32, (C, C * C), 0)
    jj = lax.broadcasted_iota(jnp.int32, (C, C * C), 1)
    expand = (jj // C == cc).astype(jnp.float32)
    jj2 = lax.broadcasted_iota(jnp.int32, (C * C, C), 0)
    dd = lax.broadcasted_iota(jnp.int32, (C * C, C), 1)
    reduce = (jj2 % C == dd).astype(jnp.float32)
    hrep = jnp.dot(hs, expand, preferred_element_type=jnp.float32)
    msg_ref[...] = jnp.dot(w * hrep, reduce, preferred_element_type=jnp.float32)


def _edge_stage(eaT, h_src, W_e, b_e2):
    grid = (EP // BE,)
    return pl.pallas_call(
        _edge_body,
        grid=grid,
        in_specs=[
            pl.BlockSpec((D_EDGE, BE), lambda i: (0, i)),
            pl.BlockSpec((BE, C), lambda i: (i, 0)),
            pl.BlockSpec((D_EDGE, C * C), lambda i: (0, 0)),
            pl.BlockSpec((1, C * C), lambda i: (0, 0)),
        ],
        out_specs=pl.BlockSpec((BE, C), lambda i: (i, 0)),
        out_shape=jax.ShapeDtypeStruct((EP, C), jnp.float32),
    )(eaT, h_src, W_e, b_e2)


def _scatter_body(msg_hbm, dst_hbm, out_hbm, idx_v, msg_v, zero_v, aggr_sh, sem,
                  sem_idx):
    cid = lax.axis_index("c")
    sid = lax.axis_index("s")
    wid = sid * NC + cid

    def idx_fire(it, carry):
        for i in range(4):
            j = it * 4 + i
            pltpu.async_copy(dst_hbm.at[pl.ds(wid * BPW + j * CH, CH)],
                             idx_v.at[j], sem_idx)
        return carry

    lax.fori_loop(0, NCHUNK // 4, idx_fire, 0)
    pltpu.async_copy(msg_hbm.at[pl.ds(wid * BPW, BPW)], msg_v, sem)

    zv = jnp.zeros((C,), jnp.float32)

    def zbody(i, carry):
        for r in range(8):
            zero_v[i * 8 + r, :] = zv
        return carry

    lax.fori_loop(0, RPT // 8, zbody, 0)

    def idx_drain(it, carry):
        for i in range(4):
            j = it * 4 + i
            pltpu.make_async_copy(dst_hbm.at[pl.ds(wid * BPW + j * CH, CH)],
                                  idx_v.at[j], sem_idx).wait()
        return carry

    lax.fori_loop(0, NCHUNK // 4, idx_drain, 0)
    pltpu.make_async_copy(msg_hbm.at[pl.ds(0, BPW)], msg_v, sem).wait()
    pltpu.sync_copy(zero_v, aggr_sh.at[pl.ds(sid * RPT, RPT)])
    plsc.subcore_barrier()

    def fire(it, carry):
        for i in range(8):
            j = it * 8 + i
            pltpu.async_copy(msg_v.at[pl.ds(j * CH, CH)],
                             aggr_sh.at[idx_v.at[j]], sem, add=True)
        return carry

    lax.fori_loop(0, NCHUNK // 8, fire, 0)
    pltpu.make_async_copy(msg_v, aggr_sh.at[pl.ds(0, BPW)], sem).wait()
    plsc.subcore_barrier()
    pltpu.sync_copy(aggr_sh.at[pl.ds(sid * RPT, RPT)],
                    out_hbm.at[pl.ds(cid * NA + sid * RPT, RPT)])


def _final_body(p0_ref, p1_ref, hr_ref, w_out_ref, b_out_ref, out_ref):
    hfin = p0_ref[...] + p1_ref[...] + hr_ref[...]
    z = jnp.sum(hfin * w_out_ref[...], axis=1, keepdims=True) + b_out_ref[...]
    lse = jnp.maximum(z, 0.0) + jnp.log(1.0 + jnp.exp(-jnp.abs(z)))
    out_ref[...] = jnp.concatenate([-lse, z - lse], axis=1)


def _final_stage(p0, p1, hr, W_out, b_out):
    grid = (N // BN,)
    return pl.pallas_call(
        _final_body,
        grid=grid,
        in_specs=[
            pl.BlockSpec((BN, C), lambda i: (i, 0)),
            pl.BlockSpec((BN, C), lambda i: (i, 0)),
            pl.BlockSpec((BN, C), lambda i: (i, 0)),
            pl.BlockSpec((1, C), lambda i: (0, 0)),
            pl.BlockSpec((1, 1), lambda i: (0, 0)),
        ],
        out_specs=pl.BlockSpec((BN, 2), lambda i: (i, 0)),
        out_shape=jax.ShapeDtypeStruct((N, 2), jnp.float32),
    )(p0, p1, hr, W_out.reshape(1, C), b_out.reshape(1, 1))


@functools.lru_cache(maxsize=1)
def _sc_kernels():
    mesh = plsc.VectorSubcoreMesh(core_axis_name="c", subcore_axis_name="s",
                                  num_cores=NC, num_subcores=NS)
    params = pltpu.CompilerParams(use_tc_tiling_on_sc=False)
    gather = pl.kernel(
        _gather_body,
        out_type=jax.ShapeDtypeStruct((EP, C), jnp.float32),
        mesh=mesh,
        compiler_params=params,
        scratch_types=[
            pltpu.VMEM((NCHUNK, CH), jnp.int32),
            pltpu.VMEM((BPW, C), jnp.float32),
            pltpu.SemaphoreType.DMA,
            pltpu.SemaphoreType.DMA,
        ],
    )
    scatter = pl.kernel(
        _scatter_body,
        out_type=jax.ShapeDtypeStruct((NC * NA, C), jnp.float32),
        mesh=mesh,
        compiler_params=params,
        scratch_types=[
            pltpu.VMEM((NCHUNK, CH), jnp.int32),
            pltpu.VMEM((BPW, C), jnp.float32),
            pltpu.VMEM((RPT, C), jnp.float32),
            pltpu.VMEM_SHARED((NA, C), jnp.float32),
            pltpu.SemaphoreType.DMA,
            pltpu.SemaphoreType.DMA,
        ],
    )
    return gather, scatter


def kernel(x, edge_index, edge_attr, W_in, b_in, W_e, b_e, root, conv_b, W_out, b_out):
    src = edge_index[0]
    dst = edge_index[1]
    pad = EP - E
    src_p = jnp.concatenate([src, jnp.zeros((pad,), jnp.int32)])
    dst_p = jnp.concatenate([dst, jnp.full((pad,), N, jnp.int32)])

    gather_k, scatter_k = _sc_kernels()
    h, hr = _node_stage(x, W_in, b_in, root, conv_b)
    h_src = gather_k(h, src_p)
    msg = _edge_stage(edge_attr.T, h_src, W_e, b_e.reshape(1, C * C))
    partials = scatter_k(msg, dst_p)
    p0 = lax.slice(partials, (0, 0), (N, C))
    p1 = lax.slice(partials, (NA, 0), (NA + N, C))
    return _final_stage(p0, p1, hr, W_out, b_out)


# speedup vs baseline: 4.8104x; 1.1995x over previous
import functools

import jax
import jax.numpy as jnp
from jax import lax
from jax.experimental import pallas as pl
from jax.experimental.pallas import tpu as pltpu
from jax.experimental.pallas import tpu_sc as plsc

N = 10000
E = 160000
D_IN = 128
D_EDGE = 16
C = 16

NC = 2
NS = 16
NW = NC * NS

CH = 128
BPW = 5120
EP = NW * BPW
NCHUNK = BPW // CH
NA = 10240
RPT = NA // NS

BN = 2000
BE = 4096


def _lrelu(v):
    return jnp.where(v > 0, v, 0.01 * v)


def _node_body(x_ref, w_in_ref, b_in_ref, root_ref, conv_b_ref, h_ref, hr_ref):
    h = _lrelu(jnp.dot(x_ref[...], w_in_ref[...],
                       preferred_element_type=jnp.float32) + b_in_ref[...])
    h_ref[...] = h
    hr_ref[...] = jnp.dot(h, root_ref[...],
                          preferred_element_type=jnp.float32) + conv_b_ref[...]


def _node_stage(x, W_in, b_in, root, conv_b):
    grid = (N // BN,)
    return pl.pallas_call(
        _node_body,
        grid=grid,
        in_specs=[
            pl.BlockSpec((BN, D_IN), lambda i: (i, 0)),
            pl.BlockSpec((D_IN, C), lambda i: (0, 0)),
            pl.BlockSpec((1, C), lambda i: (0, 0)),
            pl.BlockSpec((C, C), lambda i: (0, 0)),
            pl.BlockSpec((1, C), lambda i: (0, 0)),
        ],
        out_specs=[
            pl.BlockSpec((BN, C), lambda i: (i, 0)),
            pl.BlockSpec((BN, C), lambda i: (i, 0)),
        ],
        out_shape=[
            jax.ShapeDtypeStruct((N, C), jnp.float32),
            jax.ShapeDtypeStruct((N, C), jnp.float32),
        ],
    )(x, W_in, b_in.reshape(1, C), root, conv_b.reshape(1, C))


def _gather_body(h_hbm, src_hbm, out_hbm, idx_v, rows_v, sem, sem_idx):
    cid = lax.axis_index("c")
    sid = lax.axis_index("s")
    wid = sid * NC + cid

    def idx_fire(it, carry):
        for i in range(4):
            j = it * 4 + i
            pltpu.async_copy(src_hbm.at[pl.ds(wid * BPW + j * CH, CH)],
                             idx_v.at[j], sem_idx)
        return carry

    lax.fori_loop(0, NCHUNK // 4, idx_fire, 0)

    def idx_drain(it, carry):
        for i in range(4):
            j = it * 4 + i
            pltpu.make_async_copy(src_hbm.at[pl.ds(wid * BPW + j * CH, CH)],
                                  idx_v.at[j], sem_idx).wait()
        return carry

    lax.fori_loop(0, NCHUNK // 4, idx_drain, 0)

    def fire(it, carry):
        for i in range(8):
            j = it * 8 + i
            pltpu.async_copy(
                h_hbm.at[idx_v.at[j]],
                rows_v.at[pl.ds(j * CH, CH)], sem)
        return carry

    lax.fori_loop(0, NCHUNK // 8, fire, 0)
    pltpu.make_async_copy(h_hbm.at[pl.ds(0, BPW)], rows_v, sem).wait()
    pltpu.sync_copy(rows_v, out_hbm.at[pl.ds(wid * BPW, BPW)])


PB = BE // 8
PW = 8 * C * C


def _edge_body(eap_ref, hsp_ref, wt_ref, bt_ref, et_ref, rt_ref, msgp_ref):
    z = jnp.dot(eap_ref[...], wt_ref[...],
                preferred_element_type=jnp.float32) + bt_ref[...]
    w = _lrelu(z)
    hrep = jnp.dot(hsp_ref[...], et_ref[...], preferred_element_type=jnp.float32)
    msgp_ref[...] = jnp.dot(w * hrep, rt_ref[...],
                            preferred_element_type=jnp.float32)


def _edge_stage(eap, hsp, Wt, bt, Et, Rt):
    grid = (EP // BE,)
    return pl.pallas_call(
        _edge_body,
        grid=grid,
        in_specs=[
            pl.BlockSpec((PB, 128), lambda i: (i, 0)),
            pl.BlockSpec((PB, 128), lambda i: (i, 0)),
            pl.BlockSpec((128, PW), lambda i: (0, 0)),
            pl.BlockSpec((1, PW), lambda i: (0, 0)),
            pl.BlockSpec((128, PW), lambda i: (0, 0)),
            pl.BlockSpec((PW, 128), lambda i: (0, 0)),
        ],
        out_specs=pl.BlockSpec((PB, 128), lambda i: (i, 0)),
        out_shape=jax.ShapeDtypeStruct((EP // 8, 128), jnp.float32),
    )(eap, hsp, Wt, bt, Et, Rt)


def _scatter_body(msg_hbm, dst_hbm, out_hbm, idx_v, msg_v, zero_v, aggr_sh, sem,
                  sem_idx):
    cid = lax.axis_index("c")
    sid = lax.axis_index("s")
    wid = sid * NC + cid

    def idx_fire(it, carry):
        for i in range(4):
            j = it * 4 + i
            pltpu.async_copy(dst_hbm.at[pl.ds(wid * BPW + j * CH, CH)],
                             idx_v.at[j], sem_idx)
        return carry

    lax.fori_loop(0, NCHUNK // 4, idx_fire, 0)
    pltpu.async_copy(msg_hbm.at[pl.ds(wid * BPW, BPW)], msg_v, sem)

    zv = jnp.zeros((C,), jnp.float32)

    def zbody(i, carry):
        for r in range(8):
            zero_v[i * 8 + r, :] = zv
        return carry

    lax.fori_loop(0, RPT // 8, zbody, 0)

    def idx_drain(it, carry):
        for i in range(4):
            j = it * 4 + i
            pltpu.make_async_copy(dst_hbm.at[pl.ds(wid * BPW + j * CH, CH)],
                                  idx_v.at[j], sem_idx).wait()
        return carry

    lax.fori_loop(0, NCHUNK // 4, idx_drain, 0)
    pltpu.make_async_copy(msg_hbm.at[pl.ds(0, BPW)], msg_v, sem).wait()
    pltpu.sync_copy(zero_v, aggr_sh.at[pl.ds(sid * RPT, RPT)])
    plsc.subcore_barrier()

    def fire(it, carry):
        for i in range(8):
            j = it * 8 + i
            pltpu.async_copy(msg_v.at[pl.ds(j * CH, CH)],
                             aggr_sh.at[idx_v.at[j]], sem, add=True)
        return carry

    lax.fori_loop(0, NCHUNK // 8, fire, 0)
    pltpu.make_async_copy(msg_v, aggr_sh.at[pl.ds(0, BPW)], sem).wait()
    plsc.subcore_barrier()
    pltpu.sync_copy(aggr_sh.at[pl.ds(sid * RPT, RPT)],
                    out_hbm.at[pl.ds(cid * NA + sid * RPT, RPT)])


def _final_body(p0_ref, p1_ref, hr_ref, w_out_ref, b_out_ref, out_ref):
    hfin = p0_ref[...] + p1_ref[...] + hr_ref[...]
    z = jnp.sum(hfin * w_out_ref[...], axis=1, keepdims=True) + b_out_ref[...]
    lse = jnp.maximum(z, 0.0) + jnp.log(1.0 + jnp.exp(-jnp.abs(z)))
    out_ref[...] = jnp.concatenate([-lse, z - lse], axis=1)


def _final_stage(p0, p1, hr, W_out, b_out):
    grid = (N // BN,)
    return pl.pallas_call(
        _final_body,
        grid=grid,
        in_specs=[
            pl.BlockSpec((BN, C), lambda i: (i, 0)),
            pl.BlockSpec((BN, C), lambda i: (i, 0)),
            pl.BlockSpec((BN, C), lambda i: (i, 0)),
            pl.BlockSpec((1, C), lambda i: (0, 0)),
            pl.BlockSpec((1, 1), lambda i: (0, 0)),
        ],
        out_specs=pl.BlockSpec((BN, 2), lambda i: (i, 0)),
        out_shape=jax.ShapeDtypeStruct((N, 2), jnp.float32),
    )(p0, p1, hr, W_out.reshape(1, C), b_out.reshape(1, 1))


@functools.lru_cache(maxsize=1)
def _sc_kernels():
    mesh = plsc.VectorSubcoreMesh(core_axis_name="c", subcore_axis_name="s",
                                  num_cores=NC, num_subcores=NS)
    params = pltpu.CompilerParams(use_tc_tiling_on_sc=False)
    gather = pl.kernel(
        _gather_body,
        out_type=jax.ShapeDtypeStruct((EP, C), jnp.float32),
        mesh=mesh,
        compiler_params=params,
        scratch_types=[
            pltpu.VMEM((NCHUNK, CH), jnp.int32),
            pltpu.VMEM((BPW, C), jnp.float32),
            pltpu.SemaphoreType.DMA,
            pltpu.SemaphoreType.DMA,
        ],
    )
    scatter = pl.kernel(
        _scatter_body,
        out_type=jax.ShapeDtypeStruct((NC * NA, C), jnp.float32),
        mesh=mesh,
        compiler_params=params,
        scratch_types=[
            pltpu.VMEM((NCHUNK, CH), jnp.int32),
            pltpu.VMEM((BPW, C), jnp.float32),
            pltpu.VMEM((RPT, C), jnp.float32),
            pltpu.VMEM_SHARED((NA, C), jnp.float32),
            pltpu.SemaphoreType.DMA,
            pltpu.SemaphoreType.DMA,
        ],
    )
    return gather, scatter


def kernel(x, edge_index, edge_attr, W_in, b_in, W_e, b_e, root, conv_b, W_out, b_out):
    src = edge_index[0]
    dst = edge_index[1]
    pad = EP - E
    src_p = jnp.concatenate([src, jnp.zeros((pad,), jnp.int32)])
    dst_p = jnp.concatenate([dst, jnp.full((pad,), N, jnp.int32)])

    eye8 = jnp.eye(8, dtype=jnp.float32)
    cc = lax.broadcasted_iota(jnp.int32, (C, C * C), 0)
    jj = lax.broadcasted_iota(jnp.int32, (C, C * C), 1)
    expand = (jj // C == cc).astype(jnp.float32)
    jj2 = lax.broadcasted_iota(jnp.int32, (C * C, C), 0)
    dd = lax.broadcasted_iota(jnp.int32, (C * C, C), 1)
    reduce = (jj2 % C == dd).astype(jnp.float32)
    Wt = jnp.kron(eye8, W_e)
    bt = jnp.tile(b_e, 8).reshape(1, PW)
    Et = jnp.kron(eye8, expand)
    Rt = jnp.kron(eye8, reduce)

    gather_k, scatter_k = _sc_kernels()
    h, hr = _node_stage(x, W_in, b_in, root, conv_b)
    h_src = gather_k(h, src_p)
    eap = edge_attr.reshape(E // 8, 128)
    hsp = h_src.reshape(EP // 8, 128)
    msgp = _edge_stage(eap, hsp, Wt, bt, Et, Rt)
    msg = msgp.reshape(EP, C)
    partials = scatter_k(msg, dst_p)
    p0 = lax.slice(partials, (0, 0), (N, C))
    p1 = lax.slice(partials, (NA, 0), (NA + N, C))
    return _final_stage(p0, p1, hr, W_out, b_out)
